# Initial kernel scaffold; baseline (speedup 1.0000x reference)
import jax
import jax.numpy as jnp
from jax.experimental import pallas as pl


def kernel(loc_feat, geo_edge_index, trans_edge_index, trans_w, W1_0, b1_0, W2_0, W1_1, b1_1, W2_1):
    raise NotImplementedError("write your pallas kernel here")


# speedup vs baseline: 3.0270x; 3.0270x over previous
import functools

import jax
import jax.numpy as jnp
from jax import lax
from jax.experimental import pallas as pl
from jax.experimental.pallas import tpu as pltpu
from jax.experimental.pallas import tpu_sc as plsc

NN = 10000
DD = 256
DH = 128
HH = 1024
EE = 160000
NC = 2
NS = 16
NP = 10240
RPT = NP // NS
KE = 80
EPT = EE // NS
NCH = EPT // KE
NZ = RPT // KE
EPW = EE // (NC * NS)
KD = 40

@functools.cache
def _mesh():
  return plsc.VectorSubcoreMesh(
      core_axis_name="c", subcore_axis_name="s", num_cores=NC, num_subcores=NS)


def _agg_body(do_deg, x2, src_g, dst_g, src_t, dst_t, w_t,
              gsum2, tsum2, deg_out,
              acc_sh, idx_v, dst_v, rows_v, w_v,
              zer_v, dstd_v, ones_v, sem):
  cid = lax.axis_index("c")
  sid = lax.axis_index("s")
  ebase = sid * EPT
  rbase = sid * RPT

  zeros16 = jnp.zeros((16,), jnp.float32)
  ones16 = jnp.ones((16,), jnp.float32)

  def memset_row(e, _):
    for j in range(DH // 16):
      zer_v[e, pl.ds(j * 16, 16)] = zeros16
    return 0

  lax.fori_loop(0, KE, memset_row, 0)
  if do_deg:
    def memset_ones(e, _):
      for j in range(DH // 16):
        ones_v[e, pl.ds(j * 16, 16)] = ones16
      return 0
    lax.fori_loop(0, KD, memset_ones, 0)

  def zero_acc():
    for i in range(NZ):
      pltpu.sync_copy(zer_v, acc_sh.at[pl.ds(rbase + i * KE, KE)])

  zero_acc()
  plsc.subcore_barrier()

  def geo_chunk(i, _):
    eoff = ebase + i * KE
    pltpu.sync_copy(src_g.at[pl.ds(eoff, KE)], idx_v)
    pltpu.async_copy(x2.at[cid].at[idx_v], rows_v, sem).wait()
    pltpu.sync_copy(dst_g.at[pl.ds(eoff, KE)], dst_v)
    pltpu.sync_copy(rows_v, acc_sh.at[dst_v], add=True)
    return 0

  lax.fori_loop(0, NCH, geo_chunk, 0)
  plsc.subcore_barrier()

  pltpu.sync_copy(acc_sh.at[pl.ds(rbase, RPT)], gsum2.at[cid].at[pl.ds(rbase, RPT)])
  plsc.subcore_barrier()

  zero_acc()
  plsc.subcore_barrier()

  if do_deg:
    def deg_chunk(i, _):
      eoff = (cid * NS + sid) * EPW + i * KD
      pltpu.sync_copy(dst_g.at[pl.ds(eoff, KD)], dstd_v)
      pltpu.sync_copy(ones_v, acc_sh.at[dstd_v], add=True)
      return 0

    lax.fori_loop(0, EPW // KD, deg_chunk, 0)
    plsc.subcore_barrier()
    pltpu.sync_copy(acc_sh.at[pl.ds(rbase, RPT)],
                    deg_out.at[cid].at[pl.ds(rbase, RPT)])
    plsc.subcore_barrier()
    zero_acc()
    plsc.subcore_barrier()

  def trans_chunk(i, _):
    eoff = ebase + i * KE
    pltpu.sync_copy(src_t.at[pl.ds(eoff, KE)], idx_v)
    pltpu.async_copy(x2.at[cid].at[idx_v], rows_v, sem).wait()
    pltpu.sync_copy(w_t.at[pl.ds(eoff, KE)], w_v)

    def scale_group(g, _):
      e0 = g * 16
      w16 = w_v[pl.ds(e0, 16)]
      for lane in range(16):
        w = w16[lane]
        for j in range(DH // 16):
          rows_v[e0 + lane, pl.ds(j * 16, 16)] = (
              rows_v[e0 + lane, pl.ds(j * 16, 16)] * w)
      return 0

    lax.fori_loop(0, KE // 16, scale_group, 0)
    pltpu.sync_copy(dst_t.at[pl.ds(eoff, KE)], dst_v)
    pltpu.sync_copy(rows_v, acc_sh.at[dst_v], add=True)
    return 0

  lax.fori_loop(0, NCH, trans_chunk, 0)
  plsc.subcore_barrier()

  pltpu.sync_copy(acc_sh.at[pl.ds(rbase, RPT)], tsum2.at[cid].at[pl.ds(rbase, RPT)])


def _make_agg(do_deg):
  out_type = [
      jax.ShapeDtypeStruct((NC, NP, DH), jnp.float32),
      jax.ShapeDtypeStruct((NC, NP, DH), jnp.float32),
      jax.ShapeDtypeStruct((NC, NP, DH), jnp.float32),
  ]
  if not do_deg:
    out_type = out_type[:2]
  scratch = [
      pltpu.VMEM_SHARED((NP, DH), jnp.float32),
      pltpu.VMEM((KE,), jnp.int32),
      pltpu.VMEM((KE,), jnp.int32),
      pltpu.VMEM((KE, DH), jnp.float32),
      pltpu.VMEM((KE,), jnp.float32),
      pltpu.VMEM((KE, DH), jnp.float32),
      pltpu.VMEM((KD,), jnp.int32),
      pltpu.VMEM((KD, DH), jnp.float32),
      pltpu.SemaphoreType.DMA,
  ]

  if do_deg:
    def body(x2, src_g, dst_g, src_t, dst_t, w_t, gsum2, tsum2, deg_out,
             *scr):
      _agg_body(True, x2, src_g, dst_g, src_t, dst_t, w_t,
                gsum2, tsum2, deg_out, *scr)
  else:
    def body(x2, src_g, dst_g, src_t, dst_t, w_t, gsum2, tsum2, *scr):
      _agg_body(False, x2, src_g, dst_g, src_t, dst_t, w_t,
                gsum2, tsum2, None, *scr)

  return pl.kernel(body, out_type=out_type, mesh=_mesh(),
                   scratch_types=scratch, name="sc_agg")


_agg_deg = lambda *a: _make_agg_cached(True)(*a)
_agg = lambda *a: _make_agg_cached(False)(*a)
_make_agg_cached = functools.cache(_make_agg)

TT = 400
GRID = NN // TT


def _dense_body(x_lo, x_hi, g_lo, g_hi, t_lo, t_hi, deg_a, deg_b, w1, b1,
                geo_out, hsum_out):
  i = pl.program_id(0)
  x = jnp.concatenate([x_lo[0], x_hi[0]], axis=1)
  gs = jnp.concatenate([g_lo[0], g_hi[0]], axis=1)
  ts = jnp.concatenate([t_lo[0], t_hi[0]], axis=1)
  invd = 1.0 / (deg_a[0, :, 0:1] + deg_b[0, :, 0:1] + 1.0)
  geo = (gs + x) * invd
  geo_out[...] = geo
  hg = jnp.tanh(jnp.dot(geo, w1[...], preferred_element_type=jnp.float32)
                + b1[...])
  ht = jnp.tanh(jnp.dot(ts, w1[...], preferred_element_type=jnp.float32)
                + b1[...])
  s = jnp.concatenate([jnp.sum(hg, 0, keepdims=True),
                       jnp.sum(ht, 0, keepdims=True)], axis=0)

  @pl.when(i == 0)
  def _():
    hsum_out[...] = s

  @pl.when(i > 0)
  def _():
    hsum_out[...] += s


def _dense(x2, gsum2, tsum2, deg2, w1, b1r):
  half = lambda c: pl.BlockSpec((1, TT, DH), lambda i, c=c: (c, i, 0))
  return pl.pallas_call(
      _dense_body,
      grid=(GRID,),
      in_specs=[half(0), half(1), half(0), half(1), half(0), half(1),
                half(0), half(1),
                pl.BlockSpec((DD, HH), lambda i: (0, 0)),
                pl.BlockSpec((1, HH), lambda i: (0, 0))],
      out_specs=[pl.BlockSpec((TT, DD), lambda i: (i, 0)),
                 pl.BlockSpec((2, HH), lambda i: (0, 0))],
      out_shape=[jax.ShapeDtypeStruct((NN, DD), jnp.float32),
                 jax.ShapeDtypeStruct((2, HH), jnp.float32)],
  )(x2, x2, gsum2, gsum2, tsum2, tsum2, deg2, deg2, w1, b1r)


def _combine_body(hsum, w2r, geo, t_lo, t_hi, x2n_out):
  wm = jnp.sum(hsum[...] * w2r[...], axis=1) / NN
  m = jnp.max(wm)
  e = jnp.exp(wm - m)
  beta = e / jnp.sum(e)
  g = geo[...]
  x2n_out[0] = beta[0] * g[:, :DH] + beta[1] * t_lo[0]
  x2n_out[1] = beta[0] * g[:, DH:] + beta[1] * t_hi[0]


def _combine(hsum, w2r, geo, tsum2):
  half = lambda c: pl.BlockSpec((1, TT, DH), lambda i, c=c: (c, i, 0))
  return pl.pallas_call(
      _combine_body,
      grid=(GRID,),
      in_specs=[pl.BlockSpec((2, HH), lambda i: (0, 0)),
                pl.BlockSpec((1, HH), lambda i: (0, 0)),
                pl.BlockSpec((TT, DD), lambda i: (i, 0)),
                half(0), half(1)],
      out_specs=pl.BlockSpec((2, TT, DH), lambda i: (0, i, 0)),
      out_shape=jax.ShapeDtypeStruct((2, NN, DH), jnp.float32),
  )(hsum, w2r, geo, tsum2, tsum2)


def kernel(loc_feat, geo_edge_index, trans_edge_index, trans_w,
           W1_0, b1_0, W2_0, W1_1, b1_1, W2_1):
  src_g = geo_edge_index[0]
  dst_g = geo_edge_index[1]
  src_t = trans_edge_index[0]
  dst_t = trans_edge_index[1]
  x2 = jnp.stack([loc_feat[:, :DH], loc_feat[:, DH:]])
  b1_0r = b1_0.reshape(1, HH)
  b1_1r = b1_1.reshape(1, HH)
  w2_0r = W2_0.reshape(1, HH)
  w2_1r = W2_1.reshape(1, HH)

  gsum2, tsum2, deg2 = _agg_deg(x2, src_g, dst_g, src_t, dst_t, trans_w)
  geo1, hsum1 = _dense(x2, gsum2, tsum2, deg2, W1_0, b1_0r)
  x2 = _combine(hsum1, w2_0r, geo1, tsum2)

  gsum2, tsum2 = _agg(x2, src_g, dst_g, src_t, dst_t, trans_w)
  geo2, hsum2 = _dense(x2, gsum2, tsum2, deg2, W1_1, b1_1r)
  x2 = _combine(hsum2, w2_1r, geo2, tsum2)

  return jnp.moveaxis(x2, 0, 1).reshape(NN, DD)


# speedup vs baseline: 6.0897x; 2.0118x over previous
import functools

import jax
import jax.numpy as jnp
from jax import lax
from jax.experimental import pallas as pl
from jax.experimental.pallas import tpu as pltpu
from jax.experimental.pallas import tpu_sc as plsc

NN = 10000
DD = 256
DH = 128
HH = 1024
EE = 160000
NC = 2
NS = 16
NP = 10240
RPT = NP // NS
KE = 80
EPT = EE // NS
NCH = EPT // KE
NZ = RPT // KE
EPW = EE // (NC * NS)
KD = 40
KC = 128
EP = 163840
CPT = EP // (NS * KC)
NBUF = 2
BCH = 40

@functools.cache
def _mesh():
  return plsc.VectorSubcoreMesh(
      core_axis_name="c", subcore_axis_name="s", num_cores=NC, num_subcores=NS)


def _agg_body(do_deg, x2, src_g2, dst_g2, src_t2, dst_t2, w_t2,
              gsum2, tsum2, deg_out,
              acc_sh, idx_blk, dst_blk, w_blk,
              r0, r1, sg0, sg1, ss0, ss1):
  rows = (r0, r1)
  semg = (sg0, sg1)
  sems = (ss0, ss1)
  cid = lax.axis_index("c")
  sid = lax.axis_index("s")
  rbase = sid * RPT
  cbase = sid * CPT
  xh = x2.at[cid]

  def fill(buf, val):
    v = jnp.full((16,), val, jnp.float32)

    def row(e, _):
      for j in range(DH // 16):
        buf[e, pl.ds(j * 16, 16)] = v
      return 0

    lax.fori_loop(0, KC, row, 0)

  def zero_acc():
    fill(rows[0], 0.0)
    for i in range(RPT // KC):
      pltpu.sync_copy(rows[0], acc_sh.at[pl.ds(rbase + i * KC, KC)])

  def flush(out):
    pltpu.sync_copy(acc_sh.at[pl.ds(rbase, RPT)],
                    out.at[cid].at[pl.ds(rbase, RPT)])

  def scale_buf(buf, c):
    def group(g, _):
      w16 = w_blk[c, pl.ds(g * 16, 16)]
      for lane in range(16):
        w = w16[lane]
        for j in range(DH // 16):
          buf[g * 16 + lane, pl.ds(j * 16, 16)] = (
              buf[g * 16 + lane, pl.ds(j * 16, 16)] * w)
      return 0

    lax.fori_loop(0, KC // 16, group, 0)

  def wait_gather(b):
    pltpu.make_async_copy(xh.at[idx_blk.at[0]], rows[b], semg[b]).wait()

  def wait_scatter(b):
    pltpu.make_async_copy(rows[b], acc_sh.at[dst_blk.at[0]], sems[b]).wait()

  def run_pass(src2, dst2, scale):
    for blk in range(CPT // BCH):
      cb = cbase + blk * BCH
      pltpu.sync_copy(src2.at[pl.ds(cb, BCH)], idx_blk)
      pltpu.sync_copy(dst2.at[pl.ds(cb, BCH)], dst_blk)
      if scale:
        pltpu.sync_copy(w_t2.at[pl.ds(cb, BCH)], w_blk)

      pltpu.async_copy(xh.at[idx_blk.at[0]], rows[0], semg[0])

      def pair(p, _):
        for b in range(NBUF):
          c = p * NBUF + b
          wait_gather(b)
          if scale:
            scale_buf(rows[b], c)
          pltpu.async_copy(rows[b], acc_sh.at[dst_blk.at[c]], sems[b],
                           add=True)
          cp = c + 1
          bp = 1 - b

          @pl.when(cp < BCH)
          def _():
            @pl.when(cp >= NBUF)
            def _():
              wait_scatter(bp)
            pltpu.async_copy(xh.at[idx_blk.at[cp]], rows[bp], semg[bp])
        return 0

      lax.fori_loop(0, BCH // NBUF, pair, 0)
      for b in range(NBUF):
        wait_scatter(b)

  zero_acc()
  plsc.subcore_barrier()

  run_pass(src_g2, dst_g2, False)
  plsc.subcore_barrier()
  flush(gsum2)
  plsc.subcore_barrier()
  zero_acc()
  plsc.subcore_barrier()

  if do_deg:
    fill(rows[1], 1.0)
    for blk in range(CPT // BCH // 2):
      cb = cbase + (cid * (CPT // BCH // 2) + blk) * BCH
      pltpu.sync_copy(dst_g2.at[pl.ds(cb, BCH)], dst_blk)

      def dchunk(i, _):
        @pl.when(i >= 2)
        def _():
          wait_scatter(1)

        pltpu.async_copy(rows[1], acc_sh.at[dst_blk.at[i]], sems[1],
                         add=True)
        return 0

      lax.fori_loop(0, BCH, dchunk, 0)
      wait_scatter(1)
      wait_scatter(1)
    plsc.subcore_barrier()
    flush(deg_out)
    plsc.subcore_barrier()
    zero_acc()
    plsc.subcore_barrier()

  run_pass(src_t2, dst_t2, True)
  plsc.subcore_barrier()
  flush(tsum2)


def _make_agg(do_deg):
  out_type = [
      jax.ShapeDtypeStruct((NC, NP, DH), jnp.float32),
      jax.ShapeDtypeStruct((NC, NP, DH), jnp.float32),
      jax.ShapeDtypeStruct((NC, NP, DH), jnp.float32),
  ]
  if not do_deg:
    out_type = out_type[:2]
  scratch = (
      [pltpu.VMEM_SHARED((NP, DH), jnp.float32)]
      + [pltpu.VMEM((BCH, KC), jnp.int32)] * 2
      + [pltpu.VMEM((BCH, KC), jnp.float32)]
      + [pltpu.VMEM((KC, DH), jnp.float32)] * NBUF
      + [pltpu.SemaphoreType.DMA] * (2 * NBUF)
  )

  if do_deg:
    def body(x2, src_g, dst_g, src_t, dst_t, w_t, gsum2, tsum2, deg_out,
             *scr):
      _agg_body(True, x2, src_g, dst_g, src_t, dst_t, w_t,
                gsum2, tsum2, deg_out, *scr)
  else:
    def body(x2, src_g, dst_g, src_t, dst_t, w_t, gsum2, tsum2, *scr):
      _agg_body(False, x2, src_g, dst_g, src_t, dst_t, w_t,
                gsum2, tsum2, None, *scr)

  return pl.kernel(body, out_type=out_type, mesh=_mesh(),
                   scratch_types=scratch, name="sc_agg")


_agg_deg = lambda *a: _make_agg_cached(True)(*a)
_agg = lambda *a: _make_agg_cached(False)(*a)
_make_agg_cached = functools.cache(_make_agg)

TT = 400
GRID = NN // TT


def _dense_body(x_lo, x_hi, g_lo, g_hi, t_lo, t_hi, deg_a, deg_b, w1, b1,
                geo_out, hsum_out):
  i = pl.program_id(0)
  x = jnp.concatenate([x_lo[0], x_hi[0]], axis=1)
  gs = jnp.concatenate([g_lo[0], g_hi[0]], axis=1)
  ts = jnp.concatenate([t_lo[0], t_hi[0]], axis=1)
  invd = 1.0 / (deg_a[0, :, 0:1] + deg_b[0, :, 0:1] + 1.0)
  geo = (gs + x) * invd
  geo_out[...] = geo
  hg = jnp.tanh(jnp.dot(geo, w1[...], preferred_element_type=jnp.float32)
                + b1[...])
  ht = jnp.tanh(jnp.dot(ts, w1[...], preferred_element_type=jnp.float32)
                + b1[...])
  s = jnp.concatenate([jnp.sum(hg, 0, keepdims=True),
                       jnp.sum(ht, 0, keepdims=True)], axis=0)

  @pl.when(i == 0)
  def _():
    hsum_out[...] = s

  @pl.when(i > 0)
  def _():
    hsum_out[...] += s


def _dense(x2, gsum2, tsum2, deg2, w1, b1r):
  half = lambda c: pl.BlockSpec((1, TT, DH), lambda i, c=c: (c, i, 0))
  return pl.pallas_call(
      _dense_body,
      grid=(GRID,),
      in_specs=[half(0), half(1), half(0), half(1), half(0), half(1),
                half(0), half(1),
                pl.BlockSpec((DD, HH), lambda i: (0, 0)),
                pl.BlockSpec((1, HH), lambda i: (0, 0))],
      out_specs=[pl.BlockSpec((TT, DD), lambda i: (i, 0)),
                 pl.BlockSpec((2, HH), lambda i: (0, 0))],
      out_shape=[jax.ShapeDtypeStruct((NN, DD), jnp.float32),
                 jax.ShapeDtypeStruct((2, HH), jnp.float32)],
  )(x2, x2, gsum2, gsum2, tsum2, tsum2, deg2, deg2, w1, b1r)


def _combine_body(hsum, w2r, geo, t_lo, t_hi, x2n_out):
  wm = jnp.sum(hsum[...] * w2r[...], axis=1) / NN
  m = jnp.max(wm)
  e = jnp.exp(wm - m)
  beta = e / jnp.sum(e)
  g = geo[...]
  x2n_out[0] = beta[0] * g[:, :DH] + beta[1] * t_lo[0]
  x2n_out[1] = beta[0] * g[:, DH:] + beta[1] * t_hi[0]


def _combine(hsum, w2r, geo, tsum2):
  half = lambda c: pl.BlockSpec((1, TT, DH), lambda i, c=c: (c, i, 0))
  return pl.pallas_call(
      _combine_body,
      grid=(GRID,),
      in_specs=[pl.BlockSpec((2, HH), lambda i: (0, 0)),
                pl.BlockSpec((1, HH), lambda i: (0, 0)),
                pl.BlockSpec((TT, DD), lambda i: (i, 0)),
                half(0), half(1)],
      out_specs=pl.BlockSpec((2, TT, DH), lambda i: (0, i, 0)),
      out_shape=jax.ShapeDtypeStruct((2, NN, DH), jnp.float32),
  )(hsum, w2r, geo, tsum2, tsum2)


def kernel(loc_feat, geo_edge_index, trans_edge_index, trans_w,
           W1_0, b1_0, W2_0, W1_1, b1_1, W2_1):
  npad = EP - EE
  pad_src = jnp.arange(npad, dtype=jnp.int32) % NN
  pad_dst = NN + jnp.arange(npad, dtype=jnp.int32) % (NP - NN)

  def prep(ei):
    s = jnp.concatenate([ei[0], pad_src]).reshape(EP // KC, KC)
    d = jnp.concatenate([ei[1], pad_dst]).reshape(EP // KC, KC)
    return s, d

  src_g2, dst_g2 = prep(geo_edge_index)
  src_t2, dst_t2 = prep(trans_edge_index)
  w_t2 = jnp.concatenate(
      [trans_w, jnp.zeros((npad,), jnp.float32)]).reshape(EP // KC, KC)
  x2 = jnp.stack([loc_feat[:, :DH], loc_feat[:, DH:]])
  b1_0r = b1_0.reshape(1, HH)
  b1_1r = b1_1.reshape(1, HH)
  w2_0r = W2_0.reshape(1, HH)
  w2_1r = W2_1.reshape(1, HH)

  gsum2, tsum2, deg2 = _agg_deg(x2, src_g2, dst_g2, src_t2, dst_t2, w_t2)
  geo1, hsum1 = _dense(x2, gsum2, tsum2, deg2, W1_0, b1_0r)
  x2 = _combine(hsum1, w2_0r, geo1, tsum2)

  gsum2, tsum2 = _agg(x2, src_g2, dst_g2, src_t2, dst_t2, w_t2)
  geo2, hsum2 = _dense(x2, gsum2, tsum2, deg2, W1_1, b1_1r)
  x2 = _combine(hsum2, w2_1r, geo2, tsum2)

  return jnp.moveaxis(x2, 0, 1).reshape(NN, DD)
